# baseline (speedup 1.0000x reference)
import jax
import jax.numpy as jnp
from jax import lax
from jax.experimental import pallas as pl
from jax.experimental.pallas import tpu as pltpu
from jax.experimental.pallas import tpu_sc as plsc

VOCAB = 1000000
EMBED_DIM = 64
BATCH = 16384
HIST = 50

NC = 2
NS = 16
NW = NC * NS
EPW = BATCH // NW
GRP = 2
NGRP = EPW // GRP
NBUF = 4
NLANE = 16
KREG = EMBED_DIM // NLANE

VCHUNK = 2048


def _tbody(tt_ref, out_ref):
    out_ref[...] = tt_ref[...].T


def _relayout(table_t):
    grid = (VOCAB + VCHUNK - 1) // VCHUNK
    return pl.pallas_call(
        _tbody,
        grid=(grid,),
        in_specs=[pl.BlockSpec((EMBED_DIM, VCHUNK), lambda i: (0, i))],
        out_specs=pl.BlockSpec((VCHUNK, EMBED_DIM), lambda i: (i, 0)),
        out_shape=jax.ShapeDtypeStruct((VOCAB, EMBED_DIM), jnp.float32),
    )(table_t)


def _body(idx_hbm, table_hbm, out_hbm, idx_v, rows_v, out_v, sems):
    c = lax.axis_index("c")
    s = lax.axis_index("s")
    wid = s * NC + c

    pltpu.sync_copy(idx_hbm.at[wid], idx_v)

    inv = jnp.float32(1.0 / HIST)

    def gather(g, b):
        return pltpu.make_async_copy(
            table_hbm.at[idx_v.at[g]], rows_v.at[b], sems.at[b]
        )

    for b in range(NBUF):
        gather(b, b).start()

    def outer(it, carry):
        for b in range(NBUF):
            g = it * NBUF + b
            gather(g, b).wait()
            for e in range(GRP):
                base = e * HIST
                accs = [
                    rows_v[b, base, pl.ds(k * NLANE, NLANE)] for k in range(KREG)
                ]
                for j in range(1, HIST):
                    for k in range(KREG):
                        accs[k] = accs[k] + rows_v[b, base + j, pl.ds(k * NLANE, NLANE)]
                for k in range(KREG):
                    out_v[g * GRP + e, pl.ds(k * NLANE, NLANE)] = accs[k] * inv

            @pl.when(g + NBUF < NGRP)
            def _():
                gather(g + NBUF, b).start()
        return carry

    lax.fori_loop(0, NGRP // NBUF, outer, 0)

    pltpu.sync_copy(out_v, out_hbm.at[wid])


@jax.jit
def _run(idx3, table):
    mesh = plsc.VectorSubcoreMesh(core_axis_name="c", subcore_axis_name="s")
    f = pl.kernel(
        _body,
        out_type=jax.ShapeDtypeStruct((NW, EPW, EMBED_DIM), jnp.float32),
        mesh=mesh,
        scratch_types=[
            pltpu.VMEM((NGRP, GRP * HIST), jnp.int32),
            pltpu.VMEM((NBUF, GRP * HIST, EMBED_DIM), jnp.float32),
            pltpu.VMEM((EPW, EMBED_DIM), jnp.float32),
            pltpu.SemaphoreType.DMA((NBUF,)),
        ],
        compiler_params=pltpu.CompilerParams(use_tc_tiling_on_sc=False),
    )
    return f(idx3, table)


def kernel(indices, table):
    idx3 = indices.astype(jnp.int32).reshape(NW, NGRP, GRP * HIST)
    table_rm = _relayout(table.T)
    out3 = _run(idx3, table_rm)
    return out3.reshape(BATCH, EMBED_DIM)


# speedup vs baseline: 1.5870x; 1.5870x over previous
import jax
import jax.numpy as jnp
from jax import lax
from jax.experimental import pallas as pl
from jax.experimental.pallas import tpu as pltpu
from jax.experimental.pallas import tpu_sc as plsc

VOCAB = 1000000
EMBED_DIM = 64
BATCH = 16384
HIST = 50

NC = 2
NS = 16
NW = NC * NS
EPW = BATCH // NW
GRP = 2
NGRP = EPW // GRP
NBUF = 4
NLANE = 16
KREG = EMBED_DIM // NLANE

VCHUNK = 2048
QHALF = VCHUNK // 2
NTBLK = (VOCAB + VCHUNK - 1) // VCHUNK
VOCAB_PAD = NTBLK * VCHUNK


def _tbody(tt_ref, out_ref):
    out_ref[:, 0:EMBED_DIM] = tt_ref[:, 0:QHALF].T
    out_ref[:, EMBED_DIM : 2 * EMBED_DIM] = tt_ref[:, QHALF:VCHUNK].T


def _relayout(table_t):
    return pl.pallas_call(
        _tbody,
        grid=(NTBLK,),
        in_specs=[pl.BlockSpec((EMBED_DIM, VCHUNK), lambda i: (0, i))],
        out_specs=pl.BlockSpec((QHALF, 2 * EMBED_DIM), lambda i: (i, 0)),
        out_shape=jax.ShapeDtypeStruct((VOCAB_PAD // 2, 2 * EMBED_DIM), jnp.float32),
    )(table_t)


def _remap_indices(v):
    i = v >> 11
    r = v & (VCHUNK - 1)
    h = (r >= QHALF).astype(v.dtype)
    p = r - h * QHALF
    return (i << 11) + 2 * p + h


def _body(idx_hbm, table_hbm, out_hbm, idx_v, rows_v, out_v, sems):
    c = lax.axis_index("c")
    s = lax.axis_index("s")
    wid = s * NC + c

    pltpu.sync_copy(idx_hbm.at[wid], idx_v)

    inv = jnp.float32(1.0 / HIST)

    def gather(g, b):
        return pltpu.make_async_copy(
            table_hbm.at[idx_v.at[g]], rows_v.at[b], sems.at[b]
        )

    for b in range(NBUF):
        gather(b, b).start()

    def outer(it, carry):
        for b in range(NBUF):
            g = it * NBUF + b
            gather(g, b).wait()
            for e in range(GRP):
                base = e * HIST
                accs = [
                    rows_v[b, base, pl.ds(k * NLANE, NLANE)] for k in range(KREG)
                ]
                for j in range(1, HIST):
                    for k in range(KREG):
                        accs[k] = accs[k] + rows_v[b, base + j, pl.ds(k * NLANE, NLANE)]
                for k in range(KREG):
                    out_v[g * GRP + e, pl.ds(k * NLANE, NLANE)] = accs[k] * inv

            @pl.when(g + NBUF < NGRP)
            def _():
                gather(g + NBUF, b).start()
        return carry

    lax.fori_loop(0, NGRP // NBUF, outer, 0)

    pltpu.sync_copy(out_v, out_hbm.at[wid])


@jax.jit
def _run(idx3, table):
    mesh = plsc.VectorSubcoreMesh(core_axis_name="c", subcore_axis_name="s")
    f = pl.kernel(
        _body,
        out_type=jax.ShapeDtypeStruct((NW, EPW, EMBED_DIM), jnp.float32),
        mesh=mesh,
        scratch_types=[
            pltpu.VMEM((NGRP, GRP * HIST), jnp.int32),
            pltpu.VMEM((NBUF, GRP * HIST, EMBED_DIM), jnp.float32),
            pltpu.VMEM((EPW, EMBED_DIM), jnp.float32),
            pltpu.SemaphoreType.DMA((NBUF,)),
        ],
        compiler_params=pltpu.CompilerParams(use_tc_tiling_on_sc=False),
    )
    return f(idx3, table)


def kernel(indices, table):
    idx3 = _remap_indices(indices.astype(jnp.int32)).reshape(NW, NGRP, GRP * HIST)
    table_rm = _relayout(table.T).reshape(VOCAB_PAD, EMBED_DIM)
    out3 = _run(idx3, table_rm)
    return out3.reshape(BATCH, EMBED_DIM)


# speedup vs baseline: 2.1927x; 1.3816x over previous
import jax
import jax.numpy as jnp
from jax import lax
from jax.experimental import pallas as pl
from jax.experimental.pallas import tpu as pltpu
from jax.experimental.pallas import tpu_sc as plsc

VOCAB = 1000000
EMBED_DIM = 64
BATCH = 16384
HIST = 50

NC = 2
NS = 16
NW = NC * NS
EPW = BATCH // NW
GRP = 2
NGRP = EPW // GRP
NBUF = 4
NLANE = 16
KREG = EMBED_DIM // NLANE

VCHUNK = 8192
QHALF = VCHUNK // 2
NTBLK = (VOCAB + VCHUNK - 1) // VCHUNK
VOCAB_PAD = NTBLK * VCHUNK


def _tbody(tt_ref, out_ref):
    out_ref[:, 0:EMBED_DIM] = tt_ref[:, 0:QHALF].T
    out_ref[:, EMBED_DIM : 2 * EMBED_DIM] = tt_ref[:, QHALF:VCHUNK].T


def _relayout(table_t):
    return pl.pallas_call(
        _tbody,
        grid=(NTBLK,),
        in_specs=[pl.BlockSpec((EMBED_DIM, VCHUNK), lambda i: (0, i))],
        out_specs=pl.BlockSpec((QHALF, 2 * EMBED_DIM), lambda i: (i, 0)),
        out_shape=jax.ShapeDtypeStruct((VOCAB_PAD // 2, 2 * EMBED_DIM), jnp.float32),
    )(table_t)


def _remap_indices(v):
    i = v // VCHUNK
    r = v & (VCHUNK - 1)
    h = (r >= QHALF).astype(v.dtype)
    p = r - h * QHALF
    return i * VCHUNK + 2 * p + h


def _body(idx_hbm, table_hbm, out_hbm, idx_v, rows_v, out_v, sems):
    c = lax.axis_index("c")
    s = lax.axis_index("s")
    wid = s * NC + c

    pltpu.sync_copy(idx_hbm.at[wid], idx_v)

    inv = jnp.float32(1.0 / HIST)

    def gather(g, b):
        return pltpu.make_async_copy(
            table_hbm.at[idx_v.at[g]], rows_v.at[b], sems.at[b]
        )

    for b in range(NBUF):
        gather(b, b).start()

    def outer(it, carry):
        for b in range(NBUF):
            g = it * NBUF + b
            gather(g, b).wait()
            for e in range(GRP):
                base = e * HIST
                accs = [
                    rows_v[b, base, pl.ds(k * NLANE, NLANE)] for k in range(KREG)
                ]
                for j in range(1, HIST):
                    for k in range(KREG):
                        accs[k] = accs[k] + rows_v[b, base + j, pl.ds(k * NLANE, NLANE)]
                for k in range(KREG):
                    out_v[g * GRP + e, pl.ds(k * NLANE, NLANE)] = accs[k] * inv

            @pl.when(g + NBUF < NGRP)
            def _():
                gather(g + NBUF, b).start()
        return carry

    lax.fori_loop(0, NGRP // NBUF, outer, 0)

    pltpu.sync_copy(out_v, out_hbm.at[wid])


@jax.jit
def _run(idx3, table):
    mesh = plsc.VectorSubcoreMesh(core_axis_name="c", subcore_axis_name="s")
    f = pl.kernel(
        _body,
        out_type=jax.ShapeDtypeStruct((NW, EPW, EMBED_DIM), jnp.float32),
        mesh=mesh,
        scratch_types=[
            pltpu.VMEM((NGRP, GRP * HIST), jnp.int32),
            pltpu.VMEM((NBUF, GRP * HIST, EMBED_DIM), jnp.float32),
            pltpu.VMEM((EPW, EMBED_DIM), jnp.float32),
            pltpu.SemaphoreType.DMA((NBUF,)),
        ],
        compiler_params=pltpu.CompilerParams(use_tc_tiling_on_sc=False),
    )
    return f(idx3, table)


def kernel(indices, table):
    idx3 = _remap_indices(indices.astype(jnp.int32)).reshape(NW, NGRP, GRP * HIST)
    table_rm = _relayout(table.T).reshape(VOCAB_PAD, EMBED_DIM)
    out3 = _run(idx3, table_rm)
    return out3.reshape(BATCH, EMBED_DIM)


# speedup vs baseline: 2.3407x; 1.0675x over previous
import jax
import jax.numpy as jnp
from jax import lax
from jax.experimental import pallas as pl
from jax.experimental.pallas import tpu as pltpu
from jax.experimental.pallas import tpu_sc as plsc

VOCAB = 1000000
EMBED_DIM = 64
BATCH = 16384
HIST = 50

NC = 2
NS = 16
NW = NC * NS
EPW = BATCH // NW
GRP = 2
NGRP = EPW // GRP
NBUF = 4
NLANE = 16
KREG = EMBED_DIM // NLANE

VCHUNK = 16384
QHALF = VCHUNK // 2
NTBLK = (VOCAB + VCHUNK - 1) // VCHUNK
VOCAB_PAD = NTBLK * VCHUNK


def _tbody(tt_ref, out_ref):
    out_ref[:, 0:EMBED_DIM] = tt_ref[:, 0:QHALF].T
    out_ref[:, EMBED_DIM : 2 * EMBED_DIM] = tt_ref[:, QHALF:VCHUNK].T


def _relayout(table_t):
    return pl.pallas_call(
        _tbody,
        grid=(NTBLK,),
        in_specs=[pl.BlockSpec((EMBED_DIM, VCHUNK), lambda i: (0, i))],
        out_specs=pl.BlockSpec((QHALF, 2 * EMBED_DIM), lambda i: (i, 0)),
        out_shape=jax.ShapeDtypeStruct((VOCAB_PAD // 2, 2 * EMBED_DIM), jnp.float32),
    )(table_t)


def _remap_indices(v):
    i = v // VCHUNK
    r = v & (VCHUNK - 1)
    h = (r >= QHALF).astype(v.dtype)
    p = r - h * QHALF
    return i * VCHUNK + 2 * p + h


def _body(idx_hbm, table_hbm, out_hbm, idx_v, rows_v, out_v, sems):
    c = lax.axis_index("c")
    s = lax.axis_index("s")
    wid = s * NC + c

    pltpu.sync_copy(idx_hbm.at[wid], idx_v)

    inv = jnp.float32(1.0 / HIST)

    def gather(g, b):
        return pltpu.make_async_copy(
            table_hbm.at[idx_v.at[g]], rows_v.at[b], sems.at[b]
        )

    for b in range(NBUF):
        gather(b, b).start()

    def outer(it, carry):
        for b in range(NBUF):
            g = it * NBUF + b
            gather(g, b).wait()
            for e in range(GRP):
                base = e * HIST
                accs = [
                    rows_v[b, base, pl.ds(k * NLANE, NLANE)] for k in range(KREG)
                ]
                for j in range(1, HIST):
                    for k in range(KREG):
                        accs[k] = accs[k] + rows_v[b, base + j, pl.ds(k * NLANE, NLANE)]
                for k in range(KREG):
                    out_v[g * GRP + e, pl.ds(k * NLANE, NLANE)] = accs[k] * inv

            @pl.when(g + NBUF < NGRP)
            def _():
                gather(g + NBUF, b).start()
        return carry

    lax.fori_loop(0, NGRP // NBUF, outer, 0)

    pltpu.sync_copy(out_v, out_hbm.at[wid])


@jax.jit
def _run(idx3, table):
    mesh = plsc.VectorSubcoreMesh(core_axis_name="c", subcore_axis_name="s")
    f = pl.kernel(
        _body,
        out_type=jax.ShapeDtypeStruct((NW, EPW, EMBED_DIM), jnp.float32),
        mesh=mesh,
        scratch_types=[
            pltpu.VMEM((NGRP, GRP * HIST), jnp.int32),
            pltpu.VMEM((NBUF, GRP * HIST, EMBED_DIM), jnp.float32),
            pltpu.VMEM((EPW, EMBED_DIM), jnp.float32),
            pltpu.SemaphoreType.DMA((NBUF,)),
        ],
        compiler_params=pltpu.CompilerParams(use_tc_tiling_on_sc=False),
    )
    return f(idx3, table)


def kernel(indices, table):
    idx3 = _remap_indices(indices.astype(jnp.int32)).reshape(NW, NGRP, GRP * HIST)
    table_rm = _relayout(table.T).reshape(VOCAB_PAD, EMBED_DIM)
    out3 = _run(idx3, table_rm)
    return out3.reshape(BATCH, EMBED_DIM)


# speedup vs baseline: 2.4238x; 1.0355x over previous
import jax
import jax.numpy as jnp
from jax import lax
from jax.experimental import pallas as pl
from jax.experimental.pallas import tpu as pltpu
from jax.experimental.pallas import tpu_sc as plsc

VOCAB = 1000000
EMBED_DIM = 64
BATCH = 16384
HIST = 50

NC = 2
NS = 16
NW = NC * NS
EPW = BATCH // NW
GRP = 2
NGRP = EPW // GRP
NBUF = 4
NLANE = 16
KREG = EMBED_DIM // NLANE

VCHUNK = 32768
QHALF = VCHUNK // 2
NTBLK = (VOCAB + VCHUNK - 1) // VCHUNK
VOCAB_PAD = NTBLK * VCHUNK


def _tbody(tt_ref, out_ref):
    out_ref[:, 0:EMBED_DIM] = tt_ref[:, 0:QHALF].T
    out_ref[:, EMBED_DIM : 2 * EMBED_DIM] = tt_ref[:, QHALF:VCHUNK].T


def _relayout(table_t):
    return pl.pallas_call(
        _tbody,
        grid=(NTBLK,),
        in_specs=[pl.BlockSpec((EMBED_DIM, VCHUNK), lambda i: (0, i))],
        out_specs=pl.BlockSpec((QHALF, 2 * EMBED_DIM), lambda i: (i, 0)),
        out_shape=jax.ShapeDtypeStruct((VOCAB_PAD // 2, 2 * EMBED_DIM), jnp.float32),
    )(table_t)


def _remap_indices(v):
    i = v // VCHUNK
    r = v & (VCHUNK - 1)
    h = (r >= QHALF).astype(v.dtype)
    p = r - h * QHALF
    return i * VCHUNK + 2 * p + h


def _body(idx_hbm, table_hbm, out_hbm, idx_v, rows_v, out_v, sems):
    c = lax.axis_index("c")
    s = lax.axis_index("s")
    wid = s * NC + c

    pltpu.sync_copy(idx_hbm.at[wid], idx_v)

    inv = jnp.float32(1.0 / HIST)

    def gather(g, b):
        return pltpu.make_async_copy(
            table_hbm.at[idx_v.at[g]], rows_v.at[b], sems.at[b]
        )

    for b in range(NBUF):
        gather(b, b).start()

    def outer(it, carry):
        for b in range(NBUF):
            g = it * NBUF + b
            gather(g, b).wait()
            for e in range(GRP):
                base = e * HIST
                accs = [
                    rows_v[b, base, pl.ds(k * NLANE, NLANE)] for k in range(KREG)
                ]
                for j in range(1, HIST):
                    for k in range(KREG):
                        accs[k] = accs[k] + rows_v[b, base + j, pl.ds(k * NLANE, NLANE)]
                for k in range(KREG):
                    out_v[g * GRP + e, pl.ds(k * NLANE, NLANE)] = accs[k] * inv

            @pl.when(g + NBUF < NGRP)
            def _():
                gather(g + NBUF, b).start()
        return carry

    lax.fori_loop(0, NGRP // NBUF, outer, 0)

    pltpu.sync_copy(out_v, out_hbm.at[wid])


@jax.jit
def _run(idx3, table):
    mesh = plsc.VectorSubcoreMesh(core_axis_name="c", subcore_axis_name="s")
    f = pl.kernel(
        _body,
        out_type=jax.ShapeDtypeStruct((NW, EPW, EMBED_DIM), jnp.float32),
        mesh=mesh,
        scratch_types=[
            pltpu.VMEM((NGRP, GRP * HIST), jnp.int32),
            pltpu.VMEM((NBUF, GRP * HIST, EMBED_DIM), jnp.float32),
            pltpu.VMEM((EPW, EMBED_DIM), jnp.float32),
            pltpu.SemaphoreType.DMA((NBUF,)),
        ],
        compiler_params=pltpu.CompilerParams(use_tc_tiling_on_sc=False),
    )
    return f(idx3, table)


def kernel(indices, table):
    idx3 = _remap_indices(indices.astype(jnp.int32)).reshape(NW, NGRP, GRP * HIST)
    table_rm = _relayout(table.T).reshape(VOCAB_PAD, EMBED_DIM)
    out3 = _run(idx3, table_rm)
    return out3.reshape(BATCH, EMBED_DIM)


# speedup vs baseline: 2.4274x; 1.0015x over previous
import jax
import jax.numpy as jnp
from jax import lax
from jax.experimental import pallas as pl
from jax.experimental.pallas import tpu as pltpu
from jax.experimental.pallas import tpu_sc as plsc

VOCAB = 1000000
EMBED_DIM = 64
BATCH = 16384
HIST = 50

NC = 2
NS = 16
NW = NC * NS
EPW = BATCH // NW
GRP = 2
NGRP = EPW // GRP
NBUF = 4
NLANE = 16
KREG = EMBED_DIM // NLANE

VCHUNK = 32768
QHALF = VCHUNK // 2
NTBLK = (VOCAB + VCHUNK - 1) // VCHUNK
VOCAB_PAD = NTBLK * VCHUNK


def _tbody(tt_ref, out_ref):
    t1 = tt_ref[:, 0:QHALF].T
    t2 = tt_ref[:, QHALF:VCHUNK].T
    out_ref[...] = jnp.concatenate([t1, t2], axis=1)


def _relayout(table_t):
    return pl.pallas_call(
        _tbody,
        grid=(NTBLK,),
        in_specs=[pl.BlockSpec((EMBED_DIM, VCHUNK), lambda i: (0, i))],
        out_specs=pl.BlockSpec((QHALF, 2 * EMBED_DIM), lambda i: (i, 0)),
        out_shape=jax.ShapeDtypeStruct((VOCAB_PAD // 2, 2 * EMBED_DIM), jnp.float32),
    )(table_t)


def _remap_indices(v):
    i = v // VCHUNK
    r = v & (VCHUNK - 1)
    h = (r >= QHALF).astype(v.dtype)
    p = r - h * QHALF
    return i * VCHUNK + 2 * p + h


def _body(idx_hbm, table_hbm, out_hbm, idx_v, rows_v, out_v, sems):
    c = lax.axis_index("c")
    s = lax.axis_index("s")
    wid = s * NC + c

    pltpu.sync_copy(idx_hbm.at[wid], idx_v)

    inv = jnp.float32(1.0 / HIST)

    def gather(g, b):
        return pltpu.make_async_copy(
            table_hbm.at[idx_v.at[g]], rows_v.at[b], sems.at[b]
        )

    for b in range(NBUF):
        gather(b, b).start()

    def outer(it, carry):
        for b in range(NBUF):
            g = it * NBUF + b
            gather(g, b).wait()
            for e in range(GRP):
                base = e * HIST
                accs = [
                    rows_v[b, base, pl.ds(k * NLANE, NLANE)] for k in range(KREG)
                ]
                for j in range(1, HIST):
                    for k in range(KREG):
                        accs[k] = accs[k] + rows_v[b, base + j, pl.ds(k * NLANE, NLANE)]
                for k in range(KREG):
                    out_v[g * GRP + e, pl.ds(k * NLANE, NLANE)] = accs[k] * inv

            @pl.when(g + NBUF < NGRP)
            def _():
                gather(g + NBUF, b).start()
        return carry

    lax.fori_loop(0, NGRP // NBUF, outer, 0)

    pltpu.sync_copy(out_v, out_hbm.at[wid])


@jax.jit
def _run(idx3, table):
    mesh = plsc.VectorSubcoreMesh(core_axis_name="c", subcore_axis_name="s")
    f = pl.kernel(
        _body,
        out_type=jax.ShapeDtypeStruct((NW, EPW, EMBED_DIM), jnp.float32),
        mesh=mesh,
        scratch_types=[
            pltpu.VMEM((NGRP, GRP * HIST), jnp.int32),
            pltpu.VMEM((NBUF, GRP * HIST, EMBED_DIM), jnp.float32),
            pltpu.VMEM((EPW, EMBED_DIM), jnp.float32),
            pltpu.SemaphoreType.DMA((NBUF,)),
        ],
        compiler_params=pltpu.CompilerParams(use_tc_tiling_on_sc=False),
    )
    return f(idx3, table)


def kernel(indices, table):
    idx3 = _remap_indices(indices.astype(jnp.int32)).reshape(NW, NGRP, GRP * HIST)
    table_rm = _relayout(table.T).reshape(VOCAB_PAD, EMBED_DIM)
    out3 = _run(idx3, table_rm)
    return out3.reshape(BATCH, EMBED_DIM)
